# baseline (speedup 1.0000x reference)
import functools

import jax
import jax.numpy as jnp
from jax import lax
from jax.experimental import pallas as pl
from jax.experimental.pallas import tpu as pltpu
from jax.experimental.pallas import tpu_sc as plsc

_B, _M, _D = 8, 2048, 512
_C, _L = 4096, 32
_H = 256
_N = _B * _M
_LANES = 16
_NC, _NS = 2, 16
_NW = _NC * _NS
_CPW = _C // _NW
_RB = 2048


def _bf16_bits(x):
    u = lax.bitcast_convert_type(x, jnp.uint32)
    r = u + jnp.uint32(0x7FFF) + ((u >> 16) & jnp.uint32(1))
    return r & jnp.uint32(0xFFFF0000)


def _scores_body(mv_ref, w1_ref, b1_ref, w2_ref, o_ref, mvp_ref):
    x = mv_ref[...]
    h = jnp.dot(x, w1_ref[...], preferred_element_type=jnp.float32)
    h = jnp.maximum(h + b1_ref[...], 0.0)
    o_ref[...] = jnp.sum(h * w2_ref[...], axis=1, keepdims=True)
    lo = _bf16_bits(x[:, : _D // 2]) >> 16
    hi = _bf16_bits(x[:, _D // 2:])
    mvp_ref[...] = hi | lo


def _scores_call(mv, W1, b1, W2):
    return pl.pallas_call(
        _scores_body,
        grid=(_N // _RB,),
        in_specs=[
            pl.BlockSpec((_RB, _D), lambda i: (i, 0)),
            pl.BlockSpec((_D, _H), lambda i: (0, 0)),
            pl.BlockSpec((1, _H), lambda i: (0, 0)),
            pl.BlockSpec((1, _H), lambda i: (0, 0)),
        ],
        out_specs=[
            pl.BlockSpec((_RB, 1), lambda i: (i, 0)),
            pl.BlockSpec((_RB, _D // 2), lambda i: (i, 0)),
        ],
        out_shape=[
            jax.ShapeDtypeStruct((_N, 1), jnp.float32),
            jax.ShapeDtypeStruct((_N, _D // 2), jnp.uint32),
        ],
    )(mv, W1, b1.reshape(1, _H), W2.reshape(1, _H))


_NBUF = 8
_NOBUF = 4


def _sc_body(mv_hbm, sc_hbm, idx_hbm, len_hbm, out_hbm,
             scores_v, idx_v, len_v, rows_v, probs_v, obuf_v, osem, *sems):
    wid = lax.axis_index("s") * _NC + lax.axis_index("c")
    base = wid * _CPW
    pltpu.sync_copy(sc_hbm, scores_v)
    pltpu.sync_copy(idx_hbm.at[pl.ds(base * _L, _CPW * _L)], idx_v)
    pltpu.sync_copy(len_hbm.at[pl.ds(base, _CPW)], len_v)

    _CH = 16

    def _ln_of(c):
        return plsc.load_gather(len_v, [jnp.full((_LANES,), c, jnp.int32)])[0]

    def _issue(c, b):
        ln = _ln_of(c)
        for k in range(_L // _CH):

            @pl.when(k * _CH < ln)
            def _():
                pltpu.make_async_copy(
                    mv_hbm.at[idx_v.at[pl.ds(c * _L + k * _CH, _CH)]],
                    rows_v.at[b].at[pl.ds(k * _CH, _CH)], sems[b]
                ).start()

    def _wait(c, b):
        ln = _ln_of(c)
        for k in range(_L // _CH):

            @pl.when(k * _CH < ln)
            def _():
                pltpu.make_async_copy(
                    mv_hbm.at[idx_v.at[pl.ds(k * _CH, _CH)]],
                    rows_v.at[b].at[pl.ds(k * _CH, _CH)], sems[b]
                ).wait()

    def _softmax16(g):
        cvec = jnp.full((_LANES,), g * _LANES, jnp.int32) + \
            lax.iota(jnp.int32, _LANES)
        lnv = plsc.load_gather(len_v, [cvec])
        ibase = cvec * _L
        s = []
        for l in range(_L):
            iv = plsc.load_gather(idx_v, [ibase + l])
            sl = plsc.load_gather(scores_v, [iv])
            s.append(jnp.where(lnv > l, sl, -1e38))
        m = s[0]
        for l in range(1, _L):
            m = jnp.maximum(m, s[l])
        e = [jnp.exp(sl - m) for sl in s]
        z = e[0]
        for l in range(1, _L):
            z = z + e[l]
        for l in range(_L):
            plsc.store_scatter(probs_v, [ibase + l], e[l] / z)

    def _compute(c, b):
        ln = _ln_of(c)

        npair = _D // (2 * _LANES)

        pbase = c * _L

        def lbody(l, accs):
            pb = plsc.load_gather(
                probs_v, [jnp.full((_LANES,), pbase, jnp.int32) + l])
            new = [None] * (2 * npair)
            for j in range(npair):
                w = rows_v[b, l, pl.ds(j * _LANES, _LANES)]
                lo = plsc.bitcast(w << jnp.uint32(16), jnp.float32)
                hi = plsc.bitcast(w & jnp.uint32(0xFFFF0000), jnp.float32)
                new[j] = accs[j] + pb * lo
                new[npair + j] = accs[npair + j] + pb * hi
            return tuple(new)

        accs = lax.fori_loop(
            0, ln, lbody,
            tuple(jnp.zeros((_LANES,), jnp.float32) for _ in range(2 * npair)))
        slot = c & (_NOBUF - 1)

        @pl.when(c >= _NOBUF)
        def _():
            pltpu.make_async_copy(obuf_v.at[0], out_hbm.at[base], osem).wait()

        for j in range(2 * npair):
            obuf_v[slot, pl.ds(j * _LANES, _LANES)] = accs[j]
        pltpu.make_async_copy(obuf_v.at[slot], out_hbm.at[base + c],
                              osem).start()

    for b in range(_NBUF - 1):
        _issue(b, b)
    lax.fori_loop(0, _CPW // _LANES, lambda g, car: (_softmax16(g), car)[1], 0)

    def group(g, carry):
        for b in range(_NBUF):
            c = _NBUF * g + b

            @pl.when(c + _NBUF - 1 < _CPW)
            def _():
                _issue(c + _NBUF - 1, (b + _NBUF - 1) % _NBUF)

            _wait(c, b)
            _compute(c, b)
        return carry

    lax.fori_loop(0, _CPW // _NBUF, group, 0)
    for k in range(_NOBUF):
        pltpu.make_async_copy(obuf_v.at[0], out_hbm.at[base], osem).wait()


def _sc_call(mv, scores, idx_flat, lengths):
    mesh = plsc.VectorSubcoreMesh(core_axis_name="c", subcore_axis_name="s")
    f = pl.kernel(
        _sc_body,
        mesh=mesh,
        compiler_params=pltpu.CompilerParams(needs_layout_passes=False),
        out_type=jax.ShapeDtypeStruct((_C, _D), jnp.float32),
        scratch_types=[
            pltpu.VMEM((_N,), jnp.float32),
            pltpu.VMEM((_CPW * _L,), jnp.int32),
            pltpu.VMEM((_CPW,), jnp.int32),
            pltpu.VMEM((_NBUF, _L, _D // 2), jnp.uint32),
            pltpu.VMEM((_CPW * _L,), jnp.float32),
            pltpu.VMEM((_NOBUF, _D), jnp.float32),
        ] + [pltpu.SemaphoreType.DMA] * (_NBUF + 1),
    )
    return f(mv, scores, idx_flat, lengths)


def kernel(mention_vectors, concept_indices, concept_lengths, W1, b1, W2, b2):
    del b2
    mv = mention_vectors.reshape(_N, _D)
    scores, mv_packed = _scores_call(mv, W1, b1, W2)
    out = _sc_call(mv_packed, scores.reshape(_N),
                   concept_indices.reshape(_C * _L), concept_lengths)
    return out.reshape(_B, _C // _B, _D)


# speedup vs baseline: 1.1186x; 1.1186x over previous
import functools

import jax
import jax.numpy as jnp
from jax import lax
from jax.experimental import pallas as pl
from jax.experimental.pallas import tpu as pltpu
from jax.experimental.pallas import tpu_sc as plsc

_B, _M, _D = 8, 2048, 512
_C, _L = 4096, 32
_H = 256
_N = _B * _M
_LANES = 16
_NC, _NS = 2, 16
_NW = _NC * _NS
_CPW = _C // _NW
_RB = 2048


def _bf16_bits(x):
    u = lax.bitcast_convert_type(x, jnp.uint32)
    r = u + jnp.uint32(0x7FFF) + ((u >> 16) & jnp.uint32(1))
    return r & jnp.uint32(0xFFFF0000)


def _scores_body(mv_ref, w1_ref, b1_ref, w2_ref, o_ref, mvp_ref):
    x = mv_ref[...]
    h = jnp.dot(x, w1_ref[...], preferred_element_type=jnp.float32)
    h = jnp.maximum(h + b1_ref[...], 0.0)
    o_ref[...] = jnp.sum(h * w2_ref[...], axis=1, keepdims=True)
    lo = _bf16_bits(x[:, : _D // 2]) >> 16
    hi = _bf16_bits(x[:, _D // 2:])
    mvp_ref[...] = hi | lo


def _scores_call(mv, W1, b1, W2):
    return pl.pallas_call(
        _scores_body,
        grid=(_N // _RB,),
        in_specs=[
            pl.BlockSpec((_RB, _D), lambda i: (i, 0)),
            pl.BlockSpec((_D, _H), lambda i: (0, 0)),
            pl.BlockSpec((1, _H), lambda i: (0, 0)),
            pl.BlockSpec((1, _H), lambda i: (0, 0)),
        ],
        out_specs=[
            pl.BlockSpec((_RB, 1), lambda i: (i, 0)),
            pl.BlockSpec((_RB, _D // 2), lambda i: (i, 0)),
        ],
        out_shape=[
            jax.ShapeDtypeStruct((_N, 1), jnp.float32),
            jax.ShapeDtypeStruct((_N, _D // 2), jnp.uint32),
        ],
    )(mv, W1, b1.reshape(1, _H), W2.reshape(1, _H))


_NBUF = 4
_NOBUF = 4


def _sc_body(mv_hbm, sc_hbm, idx_hbm, len_hbm, out_hbm,
             scores_v, idx_v, len_v, rows_v, probs_v, obuf_v, osem, *sems):
    wid = lax.axis_index("s") * _NC + lax.axis_index("c")
    base = wid * _CPW
    pltpu.sync_copy(sc_hbm, scores_v)
    pltpu.sync_copy(idx_hbm.at[pl.ds(base * _L, _CPW * _L)], idx_v)
    pltpu.sync_copy(len_hbm.at[pl.ds(base, _CPW)], len_v)

    _CHUNKS = ((0, 16), (16, 8), (24, 8))

    def _ln_of(c):
        return plsc.load_gather(len_v, [jnp.full((_LANES,), c, jnp.int32)])[0]

    def _issue(c, b):
        ln = _ln_of(c)
        for off, sz in _CHUNKS:

            @pl.when(off < ln)
            def _():
                pltpu.make_async_copy(
                    mv_hbm.at[idx_v.at[pl.ds(c * _L + off, sz)]],
                    rows_v.at[b].at[pl.ds(off, sz)], sems[b]
                ).start()

    def _wait(c, b):
        ln = _ln_of(c)
        for off, sz in _CHUNKS:

            @pl.when(off < ln)
            def _():
                pltpu.make_async_copy(
                    mv_hbm.at[idx_v.at[pl.ds(off, sz)]],
                    rows_v.at[b].at[pl.ds(off, sz)], sems[b]
                ).wait()

    def _softmax16(g):
        cvec = jnp.full((_LANES,), g * _LANES, jnp.int32) + \
            lax.iota(jnp.int32, _LANES)
        lnv = plsc.load_gather(len_v, [cvec])
        ibase = cvec * _L
        s = []
        for l in range(_L):
            iv = plsc.load_gather(idx_v, [ibase + l])
            sl = plsc.load_gather(scores_v, [iv])
            s.append(jnp.where(lnv > l, sl, -1e38))
        m = s[0]
        for l in range(1, _L):
            m = jnp.maximum(m, s[l])
        e = [jnp.exp(sl - m) for sl in s]
        z = e[0]
        for l in range(1, _L):
            z = z + e[l]
        for l in range(_L):
            plsc.store_scatter(probs_v, [ibase + l], e[l] / z)

    def _compute(c, b):
        ln = _ln_of(c)

        npair = _D // (2 * _LANES)

        pbase = c * _L

        def lbody(l, accs):
            pb = plsc.load_gather(
                probs_v, [jnp.full((_LANES,), pbase, jnp.int32) + l])
            new = [None] * (2 * npair)
            for j in range(npair):
                w = rows_v[b, l, pl.ds(j * _LANES, _LANES)]
                lo = plsc.bitcast(w << jnp.uint32(16), jnp.float32)
                hi = plsc.bitcast(w & jnp.uint32(0xFFFF0000), jnp.float32)
                new[j] = accs[j] + pb * lo
                new[npair + j] = accs[npair + j] + pb * hi
            return tuple(new)

        accs = lax.fori_loop(
            0, ln, lbody,
            tuple(jnp.zeros((_LANES,), jnp.float32) for _ in range(2 * npair)))
        slot = c & (_NOBUF - 1)

        @pl.when(c >= _NOBUF)
        def _():
            pltpu.make_async_copy(obuf_v.at[0], out_hbm.at[base], osem).wait()

        for j in range(2 * npair):
            obuf_v[slot, pl.ds(j * _LANES, _LANES)] = accs[j]
        pltpu.make_async_copy(obuf_v.at[slot], out_hbm.at[base + c],
                              osem).start()

    for b in range(_NBUF - 1):
        _issue(b, b)
    lax.fori_loop(0, _CPW // _LANES, lambda g, car: (_softmax16(g), car)[1], 0)

    def group(g, carry):
        for b in range(_NBUF):
            c = _NBUF * g + b

            @pl.when(c + _NBUF - 1 < _CPW)
            def _():
                _issue(c + _NBUF - 1, (b + _NBUF - 1) % _NBUF)

            _wait(c, b)
            _compute(c, b)
        return carry

    lax.fori_loop(0, _CPW // _NBUF, group, 0)
    for k in range(_NOBUF):
        pltpu.make_async_copy(obuf_v.at[0], out_hbm.at[base], osem).wait()


def _sc_call(mv, scores, idx_flat, lengths):
    mesh = plsc.VectorSubcoreMesh(core_axis_name="c", subcore_axis_name="s")
    f = pl.kernel(
        _sc_body,
        mesh=mesh,
        compiler_params=pltpu.CompilerParams(needs_layout_passes=False),
        out_type=jax.ShapeDtypeStruct((_C, _D), jnp.float32),
        scratch_types=[
            pltpu.VMEM((_N,), jnp.float32),
            pltpu.VMEM((_CPW * _L,), jnp.int32),
            pltpu.VMEM((_CPW,), jnp.int32),
            pltpu.VMEM((_NBUF, _L, _D // 2), jnp.uint32),
            pltpu.VMEM((_CPW * _L,), jnp.float32),
            pltpu.VMEM((_NOBUF, _D), jnp.float32),
        ] + [pltpu.SemaphoreType.DMA] * (_NBUF + 1),
    )
    return f(mv, scores, idx_flat, lengths)


def kernel(mention_vectors, concept_indices, concept_lengths, W1, b1, W2, b2):
    del b2
    mv = mention_vectors.reshape(_N, _D)
    scores, mv_packed = _scores_call(mv, W1, b1, W2)
    out = _sc_call(mv_packed, scores.reshape(_N),
                   concept_indices.reshape(_C * _L), concept_lengths)
    return out.reshape(_B, _C // _B, _D)


# speedup vs baseline: 1.1261x; 1.0067x over previous
import functools

import jax
import jax.numpy as jnp
from jax import lax
from jax.experimental import pallas as pl
from jax.experimental.pallas import tpu as pltpu
from jax.experimental.pallas import tpu_sc as plsc

_B, _M, _D = 8, 2048, 512
_C, _L = 4096, 32
_H = 256
_N = _B * _M
_LANES = 16
_NC, _NS = 2, 16
_NW = _NC * _NS
_CPW = _C // _NW
_RB = 2048


def _bf16_bits(x):
    u = lax.bitcast_convert_type(x, jnp.uint32)
    r = u + jnp.uint32(0x7FFF) + ((u >> 16) & jnp.uint32(1))
    return r & jnp.uint32(0xFFFF0000)


def _scores_body(mv_ref, w1_ref, b1_ref, w2_ref, o_ref, mvp_ref):
    x = mv_ref[...]
    h = jnp.dot(x, w1_ref[...], preferred_element_type=jnp.float32)
    h = jnp.maximum(h + b1_ref[...], 0.0)
    o_ref[...] = jnp.sum(h * w2_ref[...], axis=1, keepdims=True)
    lo = _bf16_bits(x[:, : _D // 2]) >> 16
    hi = _bf16_bits(x[:, _D // 2:])
    mvp_ref[...] = hi | lo


def _scores_call(mv, W1, b1, W2):
    return pl.pallas_call(
        _scores_body,
        grid=(_N // _RB,),
        in_specs=[
            pl.BlockSpec((_RB, _D), lambda i: (i, 0)),
            pl.BlockSpec((_D, _H), lambda i: (0, 0)),
            pl.BlockSpec((1, _H), lambda i: (0, 0)),
            pl.BlockSpec((1, _H), lambda i: (0, 0)),
        ],
        out_specs=[
            pl.BlockSpec((_RB, 1), lambda i: (i, 0)),
            pl.BlockSpec((_RB, _D // 2), lambda i: (i, 0)),
        ],
        out_shape=[
            jax.ShapeDtypeStruct((_N, 1), jnp.float32),
            jax.ShapeDtypeStruct((_N, _D // 2), jnp.uint32),
        ],
    )(mv, W1, b1.reshape(1, _H), W2.reshape(1, _H))


_NBUF = 4
_NOBUF = 4


def _sc_body(mv_hbm, sc_hbm, idx_hbm, len_hbm, out_hbm,
             scores_v, idx_v, len_v, rows_v, probs_v, obuf_v, osem, *sems):
    wid = lax.axis_index("s") * _NC + lax.axis_index("c")
    base = wid * _CPW
    pltpu.sync_copy(sc_hbm, scores_v)
    pltpu.sync_copy(idx_hbm.at[pl.ds(base * _L, _CPW * _L)], idx_v)
    pltpu.sync_copy(len_hbm.at[pl.ds(base, _CPW)], len_v)

    _CHUNKS = ((0, 16), (16, 16))

    def _ln_of(c):
        return plsc.load_gather(len_v, [jnp.full((_LANES,), c, jnp.int32)])[0]

    def _issue(c, b):
        ln = _ln_of(c)
        for off, sz in _CHUNKS:

            @pl.when(off < ln)
            def _():
                pltpu.make_async_copy(
                    mv_hbm.at[idx_v.at[pl.ds(c * _L + off, sz)]],
                    rows_v.at[b].at[pl.ds(off, sz)], sems[b]
                ).start()

    def _wait(c, b):
        ln = _ln_of(c)
        for off, sz in _CHUNKS:

            @pl.when(off < ln)
            def _():
                pltpu.make_async_copy(
                    mv_hbm.at[idx_v.at[pl.ds(off, sz)]],
                    rows_v.at[b].at[pl.ds(off, sz)], sems[b]
                ).wait()

    def _softmax16(g):
        cvec = jnp.full((_LANES,), g * _LANES, jnp.int32) + \
            lax.iota(jnp.int32, _LANES)
        lnv = plsc.load_gather(len_v, [cvec])
        ibase = cvec * _L
        s = []
        for l in range(_L):
            iv = plsc.load_gather(idx_v, [ibase + l])
            sl = plsc.load_gather(scores_v, [iv])
            s.append(jnp.where(lnv > l, sl, -1e38))
        m = s[0]
        for l in range(1, _L):
            m = jnp.maximum(m, s[l])
        e = [jnp.exp(sl - m) for sl in s]
        z = e[0]
        for l in range(1, _L):
            z = z + e[l]
        for l in range(_L):
            plsc.store_scatter(probs_v, [ibase + l], e[l] / z)

    def _compute(c, b):
        ln = _ln_of(c)

        npair = _D // (2 * _LANES)

        pbase = c * _L

        def lbody(l, accs):
            pb = plsc.load_gather(
                probs_v, [jnp.full((_LANES,), pbase, jnp.int32) + l])
            new = [None] * (2 * npair)
            for j in range(npair):
                w = rows_v[b, l, pl.ds(j * _LANES, _LANES)]
                lo = plsc.bitcast(w << jnp.uint32(16), jnp.float32)
                hi = plsc.bitcast(w & jnp.uint32(0xFFFF0000), jnp.float32)
                new[j] = accs[j] + pb * lo
                new[npair + j] = accs[npair + j] + pb * hi
            return tuple(new)

        accs = lax.fori_loop(
            0, ln, lbody,
            tuple(jnp.zeros((_LANES,), jnp.float32) for _ in range(2 * npair)))
        slot = c & (_NOBUF - 1)

        @pl.when(c >= _NOBUF)
        def _():
            pltpu.make_async_copy(obuf_v.at[0], out_hbm.at[base], osem).wait()

        for j in range(2 * npair):
            obuf_v[slot, pl.ds(j * _LANES, _LANES)] = accs[j]
        pltpu.make_async_copy(obuf_v.at[slot], out_hbm.at[base + c],
                              osem).start()

    for b in range(_NBUF - 1):
        _issue(b, b)
    lax.fori_loop(0, _CPW // _LANES, lambda g, car: (_softmax16(g), car)[1], 0)

    def group(g, carry):
        for b in range(_NBUF):
            c = _NBUF * g + b

            @pl.when(c + _NBUF - 1 < _CPW)
            def _():
                _issue(c + _NBUF - 1, (b + _NBUF - 1) % _NBUF)

            _wait(c, b)
            _compute(c, b)
        return carry

    lax.fori_loop(0, _CPW // _NBUF, group, 0)
    for k in range(_NOBUF):
        pltpu.make_async_copy(obuf_v.at[0], out_hbm.at[base], osem).wait()


def _sc_call(mv, scores, idx_flat, lengths):
    mesh = plsc.VectorSubcoreMesh(core_axis_name="c", subcore_axis_name="s")
    f = pl.kernel(
        _sc_body,
        mesh=mesh,
        compiler_params=pltpu.CompilerParams(needs_layout_passes=False),
        out_type=jax.ShapeDtypeStruct((_C, _D), jnp.float32),
        scratch_types=[
            pltpu.VMEM((_N,), jnp.float32),
            pltpu.VMEM((_CPW * _L,), jnp.int32),
            pltpu.VMEM((_CPW,), jnp.int32),
            pltpu.VMEM((_NBUF, _L, _D // 2), jnp.uint32),
            pltpu.VMEM((_CPW * _L,), jnp.float32),
            pltpu.VMEM((_NOBUF, _D), jnp.float32),
        ] + [pltpu.SemaphoreType.DMA] * (_NBUF + 1),
    )
    return f(mv, scores, idx_flat, lengths)


def kernel(mention_vectors, concept_indices, concept_lengths, W1, b1, W2, b2):
    del b2
    mv = mention_vectors.reshape(_N, _D)
    scores, mv_packed = _scores_call(mv, W1, b1, W2)
    out = _sc_call(mv_packed, scores.reshape(_N),
                   concept_indices.reshape(_C * _L), concept_lengths)
    return out.reshape(_B, _C // _B, _D)
